# baseline (speedup 1.0000x reference)
import numpy as np

import jax
import jax.numpy as jnp
from jax.experimental import pallas as pl
from jax.sharding import Mesh, PartitionSpec as P


_BB = 4


def _labels_kernel(x_ref, cent_ref, out_ref):
    cent = cent_ref[...]
    cneg = cent * -2.0
    csq = jnp.sum(cent * cent, axis=0)[:, None]
    for i in range(_BB):
        xb = x_ref[i]
        xyneg = jax.lax.dot_general(
            cneg, xb, (((0,), (0,)), ((), ())),
            preferred_element_type=jnp.float32)
        xsq = jnp.sum(xb * xb, axis=0, keepdims=True)
        dist = (xsq + xyneg) + csq
        out_ref[i] = jnp.argmin(dist, axis=0).astype(jnp.int32)[None, :]


def _run(x, centroids):
    b, c, s = x.shape
    k = centroids.shape[1]
    return pl.pallas_call(
        _labels_kernel,
        grid=(b // _BB,),
        in_specs=[
            pl.BlockSpec((_BB, c, s), lambda i: (i, 0, 0)),
            pl.BlockSpec((c, k), lambda i: (0, 0)),
        ],
        out_specs=pl.BlockSpec((_BB, 1, s), lambda i: (i, 0, 0)),
        out_shape=jax.ShapeDtypeStruct((b, 1, s), jnp.int32),
    )(x, centroids)


def kernel(x, centroids):
    b, c, s = x.shape
    devs = jax.devices()
    if len(devs) >= 2 and b % (2 * _BB) == 0:
        mesh = Mesh(np.array(devs[:2]), ("d",))
        fn = jax.shard_map(
            _run, mesh=mesh,
            in_specs=(P("d", None, None), P(None, None)),
            out_specs=P("d", None, None), check_vma=False)
        out = fn(x, centroids)
    else:
        out = _run(x, centroids)
    return out.reshape(b, s)


# speedup vs baseline: 22.9454x; 22.9454x over previous
import numpy as np

import jax
import jax.numpy as jnp
from jax.experimental import pallas as pl
from jax.sharding import Mesh, PartitionSpec as P


_BB = 8


def _labels_kernel(x_ref, cent_ref, out_ref):
    cent = cent_ref[...]
    cneg = cent * -2.0
    csq = jnp.sum(cent * cent, axis=0)[:, None]
    for i in range(_BB):
        xb = x_ref[i]
        xyneg = jax.lax.dot_general(
            cneg, xb, (((0,), (0,)), ((), ())),
            preferred_element_type=jnp.float32)
        xsq = jnp.sum(xb * xb, axis=0, keepdims=True)
        dist = (xsq + xyneg) + csq
        out_ref[i] = jnp.argmin(dist, axis=0).astype(jnp.int32)[None, :]


_SS = 2


def _run(x, centroids):
    b, c, s = x.shape
    k = centroids.shape[1]
    sb = s // _SS
    return pl.pallas_call(
        _labels_kernel,
        grid=(b // _BB, _SS),
        in_specs=[
            pl.BlockSpec((_BB, c, sb), lambda i, j: (i, 0, j)),
            pl.BlockSpec((c, k), lambda i, j: (0, 0)),
        ],
        out_specs=pl.BlockSpec((_BB, 1, sb), lambda i, j: (i, 0, j)),
        out_shape=jax.ShapeDtypeStruct((b, 1, s), jnp.int32),
    )(x, centroids)


def kernel(x, centroids):
    b, c, s = x.shape
    return _run(x, centroids).reshape(b, s)


# speedup vs baseline: 23.4907x; 1.0238x over previous
import numpy as np

import jax
import jax.numpy as jnp
from jax.experimental import pallas as pl
from jax.sharding import Mesh, PartitionSpec as P


_BB = 16


def _labels_kernel(x_ref, cent_ref, out_ref):
    cent = cent_ref[...]
    cneg = cent * -2.0
    csq = jnp.sum(cent * cent, axis=0)[:, None]
    for i in range(_BB):
        xb = x_ref[i]
        xyneg = jax.lax.dot_general(
            cneg, xb, (((0,), (0,)), ((), ())),
            preferred_element_type=jnp.float32)
        xsq = jnp.sum(xb * xb, axis=0, keepdims=True)
        dist = (xsq + xyneg) + csq
        out_ref[i] = jnp.argmin(dist, axis=0).astype(jnp.int32)[None, :]


_SS = 1


def _run(x, centroids):
    b, c, s = x.shape
    k = centroids.shape[1]
    sb = s // _SS
    return pl.pallas_call(
        _labels_kernel,
        grid=(b // _BB, _SS),
        in_specs=[
            pl.BlockSpec((_BB, c, sb), lambda i, j: (i, 0, j)),
            pl.BlockSpec((c, k), lambda i, j: (0, 0)),
        ],
        out_specs=pl.BlockSpec((_BB, 1, sb), lambda i, j: (i, 0, j)),
        out_shape=jax.ShapeDtypeStruct((b, 1, s), jnp.int32),
    )(x, centroids)


def kernel(x, centroids):
    b, c, s = x.shape
    return _run(x, centroids).reshape(b, s)
